# baseline (speedup 1.0000x reference)
import jax
import jax.numpy as jnp
from jax import lax
from jax.experimental import pallas as pl
from jax.experimental.pallas import tpu as pltpu
from jax.experimental.pallas import tpu_sc as plsc

_INFO = plsc.get_sparse_core_info()
_NC, _NS, _L = _INFO.num_cores, _INFO.num_subcores, _INFO.num_lanes
_NW = _NC * _NS

_D4 = 512
_C = 64
_SUP = 2048
_NCH = _SUP // _C

_N_SC = 1048576
_M = 2048


def _sc_body(b0_hbm, r0_hbm, b1_hbm, r1_hbm, tab_hbm, out_hbm,
             ib0, ir0, ib1, ir1, cidx, rows0, rows1,
             gsem0, gsem1, osem0, osem1):
    cid = lax.axis_index("c")
    sid = lax.axis_index("s")
    wid = sid * _NC + cid
    n_pairs = b0_hbm.shape[0]
    per_w = n_pairs // _NW
    n_super = per_w // _SUP

    rows = (rows0, rows1)
    gsems = (gsem0, gsem1)
    osems = (osem0, osem1)

    def super_body(s_i, _):
        sup_start = wid * per_w + s_i * _SUP
        sl = pl.ds(sup_start, _SUP)
        pltpu.sync_copy(b0_hbm.at[sl], ib0)
        pltpu.sync_copy(r0_hbm.at[sl], ir0)
        pltpu.sync_copy(b1_hbm.at[sl], ib1)
        pltpu.sync_copy(r1_hbm.at[sl], ir1)

        def cvt(i, _):
            s = pl.ds(i * _L, _L)
            cidx[s] = (ib0[s] * 5 + ir0[s]) * 20 + (ib1[s] * 5 + ir1[s])
            return _

        lax.fori_loop(0, _SUP // _L, cvt, 0)

        def pair_body(p, _):
            for b in range(2):
                ch = p * 2 + b
                first_use = (s_i == 0) & (p == 0)

                @pl.when(jnp.logical_not(first_use))
                def _wait():
                    pltpu.make_async_copy(
                        rows[b], out_hbm.at[pl.ds(0, _C)], osems[b]).wait()

                pltpu.async_copy(
                    tab_hbm.at[cidx.at[pl.ds(ch * _C, _C)]],
                    rows[b], gsems[b]).wait()
                out_off = sup_start + ch * _C
                pltpu.async_copy(
                    rows[b], out_hbm.at[pl.ds(out_off, _C)], osems[b])
            return _

        lax.fori_loop(0, _NCH // 2, pair_body, 0)
        return _

    lax.fori_loop(0, n_super, super_body, 0)

    for b in range(2):
        pltpu.make_async_copy(
            rows[b], out_hbm.at[pl.ds(0, _C)], osems[b]).wait()


def _tc_body(base_ref, refi_ref, tab_ref, out_ref):
    cidx = base_ref[...] * 5 + refi_ref[...]
    iota = lax.broadcasted_iota(jnp.int32, (_M, 32), 1)
    onehot = (cidx == iota).astype(jnp.float32)
    out_ref[...] = lax.dot_general(
        onehot, tab_ref[...],
        dimension_numbers=(((1,), (0,)), ((), ())),
        preferred_element_type=jnp.float32,
    )


@jax.jit
def kernel(batch_base_seq, batch_ref_seq, read_table, ref_table):
    B, S = batch_base_seq.shape
    D = read_table.shape[1]
    N = B * S
    n_tc = N - _N_SC
    c = jnp.arange(20)
    combined = jnp.concatenate(
        [read_table[c // 5], ref_table[c % 5]], axis=1)
    cp = jnp.arange(400)
    pair_tab = jnp.concatenate(
        [combined[cp // 20], combined[cp % 20]], axis=1)
    base = batch_base_seq.astype(jnp.int32).reshape(N)
    refi = batch_ref_seq.astype(jnp.int32).reshape(N)

    sc_base = base[n_tc:].reshape(_N_SC // 2, 2)
    sc_refi = refi[n_tc:].reshape(_N_SC // 2, 2)
    b0, b1 = sc_base[:, 0], sc_base[:, 1]
    r0, r1 = sc_refi[:, 0], sc_refi[:, 1]

    sc_run = pl.kernel(
        _sc_body,
        out_type=jax.ShapeDtypeStruct((_N_SC // 2, 4 * D), jnp.float32),
        mesh=plsc.VectorSubcoreMesh(core_axis_name="c", subcore_axis_name="s"),
        scratch_types=[
            pltpu.VMEM((_SUP,), jnp.int32),
            pltpu.VMEM((_SUP,), jnp.int32),
            pltpu.VMEM((_SUP,), jnp.int32),
            pltpu.VMEM((_SUP,), jnp.int32),
            pltpu.VMEM((_SUP,), jnp.int32),
            pltpu.VMEM((_C, _D4), jnp.float32),
            pltpu.VMEM((_C, _D4), jnp.float32),
            pltpu.SemaphoreType.DMA,
            pltpu.SemaphoreType.DMA,
            pltpu.SemaphoreType.DMA,
            pltpu.SemaphoreType.DMA,
        ],
    )
    sc_out = sc_run(b0, r0, b1, r1, pair_tab)

    tab32 = jnp.pad(combined, ((0, 12), (0, 0)))
    tc_out = pl.pallas_call(
        _tc_body,
        grid=(n_tc // _M,),
        in_specs=[
            pl.BlockSpec((_M, 1), lambda i: (i, 0)),
            pl.BlockSpec((_M, 1), lambda i: (i, 0)),
            pl.BlockSpec((32, 2 * D), lambda i: (0, 0)),
        ],
        out_specs=pl.BlockSpec((_M, 2 * D), lambda i: (i, 0)),
        out_shape=jax.ShapeDtypeStruct((n_tc, 2 * D), jnp.float32),
        compiler_params=pltpu.CompilerParams(
            dimension_semantics=("parallel",)),
    )(base[:n_tc].reshape(n_tc, 1), refi[:n_tc].reshape(n_tc, 1), tab32)

    out = jnp.concatenate(
        [tc_out, sc_out.reshape(_N_SC, 2 * D)], axis=0)
    return out.reshape(B, S, 2 * D)
